# baseline (speedup 1.0000x reference)
import jax
import jax.numpy as jnp
from jax import lax
from jax.experimental import pallas as pl
from jax.experimental.pallas import tpu as pltpu

B, N, D = 4, 8192, 2048
W = 256
NB, R = 64, 128
KC = 8
NG = R // 8


def _shiftc(w, c):
    k, r, ww = w.shape
    return jnp.concatenate([c[:, None, :], w[:, : r - 1, :]], axis=1)


def _band_kernel(x_ref, rv_ref, o_ref, buf_ref):
    rv = rv_ref[0]
    rv3 = rv[None]

    def pa(kb, acc):
        a = jnp.abs(x_ref[0, kb, :, :])
        return acc + jnp.sum(a, axis=0, keepdims=True)

    s1 = lax.fori_loop(0, NB, pa, jnp.zeros((1, W), jnp.float32))

    def pb(kb, acc):
        a = jnp.abs(x_ref[0, kb, :, :])
        pi = jnp.exp((a / s1) * 2.0)
        buf_ref[kb, :, :] = pi
        p3 = pi.reshape(R // 8, 8, W)
        for m in range(R // 8):
            acc = acc + p3[m]
        return acc

    acc = lax.fori_loop(0, NB, pb, jnp.zeros((8, W), jnp.float32))
    for s in (4, 2, 1):
        acc = acc[:s] + acc[s:2 * s]
    s2 = acc

    def pb2(kb, _):
        buf_ref[kb, :, :] = buf_ref[kb, :, :] / s2
        return 0

    lax.fori_loop(0, NB, pb2, 0)

    def pc(i, acc):
        v = acc + buf_ref[:, pl.ds(i, 1), :]
        buf_ref[:, pl.ds(i, 1), :] = v
        return v

    acc = lax.fori_loop(0, R, pc, jnp.zeros((NB, 1, W), jnp.float32))
    sums = acc[:, 0, :]

    rows = [jnp.zeros((1, W), jnp.float32)]
    run = sums[0:1]
    for j in range(1, NB):
        rows.append(run)
        run = run + sums[j:j + 1]
    excl = jnp.concatenate(rows, axis=0)

    g1 = jnp.zeros((1, W), jnp.float32)
    for kc in range(NB // KC):
        exc = excl[kc * KC:(kc + 1) * KC]
        exc3 = exc[:, None, :]
        if kc == 0:
            k0 = lax.broadcasted_iota(jnp.int32, (KC, W), 0) == 0
            pc = jnp.where(k0, -1.0, exc + 0.001)
        else:
            pc = exc + 0.001
        gacc = jnp.zeros((KC, W), jnp.float32)
        for g in range(NG):
            w = buf_ref[pl.ds(kc * KC, KC), 8 * g:8 * g + 8, :]
            ct = (w + exc3) + 0.001
            prev = jnp.concatenate([pc[:, None, :], ct[:, :7, :]], axis=1)
            cross = (ct > rv3) & (prev <= rv3)
            a = jnp.abs(x_ref[0, pl.ds(kc * KC, KC), 8 * g:8 * g + 8, :])
            gacc = gacc + jnp.sum(jnp.where(cross, a, 0.0), axis=1)
            pc = ct[:, 7, :]
        g1 = g1 + jnp.sum(gacc, axis=0, keepdims=True)

    def pe(kb, _):
        v = x_ref[0, kb, :, :]
        o_ref[0, kb, :, :] = jnp.where(jnp.abs(v) >= g1, v, 0.0)
        return 0

    lax.fori_loop(0, NB, pe, 0)


def kernel(x):
    rv = jax.random.uniform(jax.random.key(42), (B, D), dtype=x.dtype)
    rv = rv.reshape(B, 1, D)
    x4 = x.reshape(B, NB, R, D)
    out = pl.pallas_call(
        _band_kernel,
        grid=(B, D // W),
        in_specs=[
            pl.BlockSpec((1, NB, R, W), lambda b, j: (b, 0, 0, j)),
            pl.BlockSpec((1, 1, W), lambda b, j: (b, 0, j)),
        ],
        out_specs=pl.BlockSpec((1, NB, R, W), lambda b, j: (b, 0, 0, j)),
        out_shape=jax.ShapeDtypeStruct((B, NB, R, D), x.dtype),
        scratch_shapes=[pltpu.VMEM((NB, R, W), jnp.float32)],
        compiler_params=pltpu.CompilerParams(
            dimension_semantics=("parallel", "parallel")),
    )(x4, rv)
    return out.reshape(B, N, D)


# speedup vs baseline: 1.3179x; 1.3179x over previous
import jax
import jax.numpy as jnp
from jax import lax
from jax.experimental import pallas as pl
from jax.experimental.pallas import tpu as pltpu

B, N, D = 4, 8192, 2048
W = 256
NB, R = 64, 128
KC = 4
NG = R // 8


def _shiftc(w, c):
    k, r, ww = w.shape
    return jnp.concatenate([c[:, None, :], w[:, : r - 1, :]], axis=1)


def _band_kernel(x_ref, rv_ref, o_ref, buf_ref):
    rv = rv_ref[0]
    rv3 = rv[None]

    def pa(kb, acc):
        a = jnp.abs(x_ref[0, kb, :, :])
        return acc + jnp.sum(a, axis=0, keepdims=True)

    s1 = lax.fori_loop(0, NB, pa, jnp.zeros((1, W), jnp.float32))

    def pb(kb, acc):
        a = jnp.abs(x_ref[0, kb, :, :])
        pi = jnp.exp((a / s1) * 2.0)
        buf_ref[kb, :, :] = pi
        p3 = pi.reshape(R // 8, 8, W)
        for m in range(R // 8):
            acc = acc + p3[m]
        return acc

    acc = lax.fori_loop(0, NB, pb, jnp.zeros((8, W), jnp.float32))
    for s in (4, 2, 1):
        acc = acc[:s] + acc[s:2 * s]
    s2 = acc
    s23 = s2[None]

    sums_parts = []
    for kc in range(NB // KC):
        carry = jnp.zeros((KC, W), jnp.float32)
        for g in range(NG):
            th = buf_ref[pl.ds(kc * KC, KC), 8 * g:8 * g + 8, :] / s23
            w = th
            for _ in range(8):
                w = th + _shiftc(w, carry)
            buf_ref[pl.ds(kc * KC, KC), 8 * g:8 * g + 8, :] = w
            carry = w[:, 7, :]
        sums_parts.append(carry)
    sums = jnp.concatenate(sums_parts, axis=0)

    rows = [jnp.zeros((1, W), jnp.float32)]
    run = sums[0:1]
    for j in range(1, NB):
        rows.append(run)
        run = run + sums[j:j + 1]
    excl = jnp.concatenate(rows, axis=0)

    g1 = jnp.zeros((1, W), jnp.float32)
    for kc in range(NB // KC):
        exc = excl[kc * KC:(kc + 1) * KC]
        exc3 = exc[:, None, :]
        if kc == 0:
            k0 = lax.broadcasted_iota(jnp.int32, (KC, W), 0) == 0
            pc = jnp.where(k0, -1.0, exc + 0.001)
        else:
            pc = exc + 0.001
        gacc = jnp.zeros((KC, W), jnp.float32)
        for g in range(NG):
            w = buf_ref[pl.ds(kc * KC, KC), 8 * g:8 * g + 8, :]
            ct = (w + exc3) + 0.001
            prev = jnp.concatenate([pc[:, None, :], ct[:, :7, :]], axis=1)
            cross = (ct > rv3) & (prev <= rv3)
            a = jnp.abs(x_ref[0, pl.ds(kc * KC, KC), 8 * g:8 * g + 8, :])
            gacc = gacc + jnp.sum(jnp.where(cross, a, 0.0), axis=1)
            pc = ct[:, 7, :]
        g1 = g1 + jnp.sum(gacc, axis=0, keepdims=True)

    def pe(kb, _):
        v = x_ref[0, kb, :, :]
        o_ref[0, kb, :, :] = jnp.where(jnp.abs(v) >= g1, v, 0.0)
        return 0

    lax.fori_loop(0, NB, pe, 0)


def kernel(x):
    rv = jax.random.uniform(jax.random.key(42), (B, D), dtype=x.dtype)
    rv = rv.reshape(B, 1, D)
    x4 = x.reshape(B, NB, R, D)
    out = pl.pallas_call(
        _band_kernel,
        grid=(B, D // W),
        in_specs=[
            pl.BlockSpec((1, NB, R, W), lambda b, j: (b, 0, 0, j)),
            pl.BlockSpec((1, 1, W), lambda b, j: (b, 0, j)),
        ],
        out_specs=pl.BlockSpec((1, NB, R, W), lambda b, j: (b, 0, 0, j)),
        out_shape=jax.ShapeDtypeStruct((B, NB, R, D), x.dtype),
        scratch_shapes=[pltpu.VMEM((NB, R, W), jnp.float32)],
        compiler_params=pltpu.CompilerParams(
            dimension_semantics=("parallel", "parallel")),
    )(x4, rv)
    return out.reshape(B, N, D)


# speedup vs baseline: 1.3257x; 1.0059x over previous
import jax
import jax.numpy as jnp
from jax import lax
from jax.experimental import pallas as pl
from jax.experimental.pallas import tpu as pltpu

B, N, D = 4, 8192, 2048
W = 256
NB, R = 64, 128
KC = 8
NG = R // 8


def _shiftc(w, c):
    k, r, ww = w.shape
    return jnp.concatenate([c[:, None, :], w[:, : r - 1, :]], axis=1)


def _band_kernel(x_ref, rv_ref, o_ref):
    rv = rv_ref[0]
    rv3 = rv[None]

    def pa(kb, acc):
        a = jnp.abs(x_ref[0, kb, :, :])
        return acc + jnp.sum(a, axis=0, keepdims=True)

    s1 = lax.fori_loop(0, NB, pa, jnp.zeros((1, W), jnp.float32))

    def pb(kb, acc):
        a = jnp.abs(x_ref[0, kb, :, :])
        pi = jnp.exp((a / s1) * 2.0)
        o_ref[0, kb, :, :] = pi
        p3 = pi.reshape(R // 8, 8, W)
        for m in range(R // 8):
            acc = acc + p3[m]
        return acc

    acc = lax.fori_loop(0, NB, pb, jnp.zeros((8, W), jnp.float32))
    for s in (4, 2, 1):
        acc = acc[:s] + acc[s:2 * s]
    s2 = acc
    s23 = s2[None]

    sums_parts = []
    for kc in range(NB // KC):
        carry = jnp.zeros((KC, W), jnp.float32)
        for g in range(NG):
            th = o_ref[0, pl.ds(kc * KC, KC), 8 * g:8 * g + 8, :] / s23
            w = th
            for _ in range(8):
                w = th + _shiftc(w, carry)
            o_ref[0, pl.ds(kc * KC, KC), 8 * g:8 * g + 8, :] = w
            carry = w[:, 7, :]
        sums_parts.append(carry)
    sums = jnp.concatenate(sums_parts, axis=0)

    rows = [jnp.zeros((1, W), jnp.float32)]
    run = sums[0:1]
    for j in range(1, NB):
        rows.append(run)
        run = run + sums[j:j + 1]
    excl = jnp.concatenate(rows, axis=0)

    g1 = jnp.zeros((1, W), jnp.float32)
    for kc in range(NB // KC):
        exc = excl[kc * KC:(kc + 1) * KC]
        exc3 = exc[:, None, :]
        if kc == 0:
            k0 = lax.broadcasted_iota(jnp.int32, (KC, W), 0) == 0
            pc = jnp.where(k0, -1.0, exc + 0.001)
        else:
            pc = exc + 0.001
        gacc = jnp.zeros((KC, W), jnp.float32)
        for g in range(NG):
            w = o_ref[0, pl.ds(kc * KC, KC), 8 * g:8 * g + 8, :]
            ct = (w + exc3) + 0.001
            prev = jnp.concatenate([pc[:, None, :], ct[:, :7, :]], axis=1)
            cross = (ct > rv3) & (prev <= rv3)
            a = jnp.abs(x_ref[0, pl.ds(kc * KC, KC), 8 * g:8 * g + 8, :])
            gacc = gacc + jnp.sum(jnp.where(cross, a, 0.0), axis=1)
            pc = ct[:, 7, :]
        g1 = g1 + jnp.sum(gacc, axis=0, keepdims=True)

    def pe(kb, _):
        v = x_ref[0, kb, :, :]
        o_ref[0, kb, :, :] = jnp.where(jnp.abs(v) >= g1, v, 0.0)
        return 0

    lax.fori_loop(0, NB, pe, 0)


def kernel(x):
    rv = jax.random.uniform(jax.random.key(42), (B, D), dtype=x.dtype)
    rv = rv.reshape(B, 1, D)
    x4 = x.reshape(B, NB, R, D)
    out = pl.pallas_call(
        _band_kernel,
        grid=(B, D // W),
        in_specs=[
            pl.BlockSpec((1, NB, R, W), lambda b, j: (b, 0, 0, j)),
            pl.BlockSpec((1, 1, W), lambda b, j: (b, 0, j)),
        ],
        out_specs=pl.BlockSpec((1, NB, R, W), lambda b, j: (b, 0, 0, j)),
        out_shape=jax.ShapeDtypeStruct((B, NB, R, D), x.dtype),
        compiler_params=pltpu.CompilerParams(
            dimension_semantics=("parallel", "parallel")),
    )(x4, rv)
    return out.reshape(B, N, D)


# speedup vs baseline: 1.3536x; 1.0211x over previous
import jax
import jax.numpy as jnp
from jax import lax
from jax.experimental import pallas as pl
from jax.experimental.pallas import tpu as pltpu

B, N, D = 4, 8192, 2048
W = 256
NB, R = 64, 128
KC = 8
NG = R // 8


def _shiftc(w, c):
    k, r, ww = w.shape
    return jnp.concatenate([c[:, None, :], w[:, : r - 1, :]], axis=1)


def _band_kernel(x_ref, rv_ref, o_ref):
    rv = rv_ref[0]
    rv3 = rv[None]

    def pa(kb, acc):
        a = jnp.abs(x_ref[0, kb, :, :])
        return acc + jnp.sum(a, axis=0, keepdims=True)

    s1 = lax.fori_loop(0, NB, pa, jnp.zeros((1, W), jnp.float32))

    def pb(kb, acc):
        a = jnp.abs(x_ref[0, pl.ds(2 * kb, 2), :, :])
        pi = jnp.exp((a / s1) * 2.0)
        o_ref[0, pl.ds(2 * kb, 2), :, :] = pi
        p3 = pi.reshape(2 * R // 8, 8, W)
        for m in range(2 * R // 8):
            acc = acc + p3[m]
        return acc

    acc = lax.fori_loop(0, NB // 2, pb, jnp.zeros((8, W), jnp.float32))
    for s in (4, 2, 1):
        acc = acc[:s] + acc[s:2 * s]
    s2 = acc
    s23 = s2[None]

    sums_parts = []
    for kc in range(NB // KC):
        carry = jnp.zeros((KC, W), jnp.float32)
        for g in range(NG):
            th = o_ref[0, pl.ds(kc * KC, KC), 8 * g:8 * g + 8, :] / s23
            w = th
            for _ in range(8):
                w = th + _shiftc(w, carry)
            o_ref[0, pl.ds(kc * KC, KC), 8 * g:8 * g + 8, :] = w
            carry = w[:, 7, :]
        sums_parts.append(carry)
    sums = jnp.concatenate(sums_parts, axis=0)

    rows = [jnp.zeros((1, W), jnp.float32)]
    run = sums[0:1]
    for j in range(1, NB):
        rows.append(run)
        run = run + sums[j:j + 1]
    excl = jnp.concatenate(rows, axis=0)

    g1 = jnp.zeros((1, W), jnp.float32)
    for kc in range(NB // KC):
        exc = excl[kc * KC:(kc + 1) * KC]
        exc3 = exc[:, None, :]
        if kc == 0:
            k0 = lax.broadcasted_iota(jnp.int32, (KC, W), 0) == 0
            pc = jnp.where(k0, -1.0, exc + 0.001)
        else:
            pc = exc + 0.001
        gacc = jnp.zeros((KC, W), jnp.float32)
        for g in range(NG):
            w = o_ref[0, pl.ds(kc * KC, KC), 8 * g:8 * g + 8, :]
            ct = (w + exc3) + 0.001
            prev = jnp.concatenate([pc[:, None, :], ct[:, :7, :]], axis=1)
            cross = (ct > rv3) & (prev <= rv3)
            a = jnp.abs(x_ref[0, pl.ds(kc * KC, KC), 8 * g:8 * g + 8, :])
            gacc = gacc + jnp.sum(jnp.where(cross, a, 0.0), axis=1)
            pc = ct[:, 7, :]
        g1 = g1 + jnp.sum(gacc, axis=0, keepdims=True)

    def pe(kb, _):
        v = x_ref[0, kb, :, :]
        o_ref[0, kb, :, :] = jnp.where(jnp.abs(v) >= g1, v, 0.0)
        return 0

    lax.fori_loop(0, NB, pe, 0)


def kernel(x):
    rv = jax.random.uniform(jax.random.key(42), (B, D), dtype=x.dtype)
    rv = rv.reshape(B, 1, D)
    x4 = x.reshape(B, NB, R, D)
    out = pl.pallas_call(
        _band_kernel,
        grid=(B, D // W),
        in_specs=[
            pl.BlockSpec((1, NB, R, W), lambda b, j: (b, 0, 0, j)),
            pl.BlockSpec((1, 1, W), lambda b, j: (b, 0, j)),
        ],
        out_specs=pl.BlockSpec((1, NB, R, W), lambda b, j: (b, 0, 0, j)),
        out_shape=jax.ShapeDtypeStruct((B, NB, R, D), x.dtype),
        compiler_params=pltpu.CompilerParams(
            dimension_semantics=("parallel", "parallel")),
    )(x4, rv)
    return out.reshape(B, N, D)
